# baseline (speedup 1.0000x reference)
import functools

import jax
import jax.numpy as jnp
from jax import lax
from jax.experimental import pallas as pl
from jax.experimental.pallas import tpu as pltpu
from jax.experimental.pallas import tpu_sc as plsc

NC = 2
NS = 16
NW = NC * NS
K = 50
L = 16


def _deg_body(np_, nq, dst_hbm, out_hbm, idx_v, ldeg_v):
    c = lax.axis_index("c")
    s = lax.axis_index("s")
    wid = s * NC + c
    zero = jnp.zeros((L,), jnp.float32)
    ones = jnp.ones((L,), jnp.float32)

    def zr(r, _):
        ldeg_v[pl.ds(r * L, L)] = zero
        return 0

    lax.fori_loop(0, np_ // L, zr, 0)
    pltpu.sync_copy(dst_hbm.at[wid], idx_v)

    def q(i, _):
        for cb in range(128 // L):
            iv = idx_v[i, pl.ds(cb * L, L)]
            plsc.addupdate_scatter(ldeg_v, [iv], ones)
        return 0

    lax.fori_loop(0, nq, q, 0)
    pltpu.sync_copy(ldeg_v, out_hbm.at[wid, 0])


CH = 64
GB = 20


def _gcn_body(np_, ab, hh, xs1_hbm, src_hbm, dst_hbm, dis_hbm, b1_hbm,
              out_hbm, xs2_hbm,
              src_v, dst_v, rows_v, rows2_v, ibuf_v, dis_v, b1_v,
              sem, sem2, acc_sh):
    c = lax.axis_index("c")
    s = lax.axis_index("s")
    pt = np_ // NS

    pltpu.sync_copy(b1_hbm.at[c], b1_v)
    for t in range(pt // CH):
        pltpu.sync_copy(xs1_hbm.at[pl.ds(c * np_ + s * pt + t * CH, CH)],
                        ibuf_v)
        pltpu.sync_copy(ibuf_v, acc_sh.at[pl.ds(s * pt + t * CH, CH)])
    plsc.subcore_barrier()

    def edge_sweep(tbl_hbm):
        bufs = (rows_v, rows2_v)
        sems = (sem, sem2)

        def grp(g, _):
            pltpu.sync_copy(src_hbm.at[c, s, g], src_v)
            pltpu.sync_copy(dst_hbm.at[s, g], dst_v)
            pend = pltpu.async_copy(tbl_hbm.at[src_v.at[0]], bufs[0], sems[0])
            for b in range(GB):
                pend.wait()
                if b + 1 < GB:
                    pend = pltpu.async_copy(tbl_hbm.at[src_v.at[b + 1]],
                                            bufs[(b + 1) % 2], sems[(b + 1) % 2])
                pltpu.sync_copy(bufs[b % 2], acc_sh.at[dst_v.at[b]], add=True)
            return 0

        lax.fori_loop(0, ab // GB, grp, 0)

    edge_sweep(xs1_hbm)
    plsc.subcore_barrier()

    for t in range(pt // CH):
        pltpu.sync_copy(acc_sh.at[pl.ds(s * pt + t * CH, CH)], ibuf_v)
        pltpu.sync_copy(dis_hbm.at[pl.ds(s * pt + t * CH, CH)], dis_v)

        def row(r, _):
            d = dis_v[r, pl.ds(0, L)]
            for cb in range(hh // L):
                v = ibuf_v[r, pl.ds(cb * L, L)]
                b = b1_v[0, pl.ds(cb * L, L)]
                ibuf_v[r, pl.ds(cb * L, L)] = jnp.maximum(d * (d * v + b), 0.0)
            return 0

        lax.fori_loop(0, CH, row, 0)
        pltpu.sync_copy(ibuf_v, acc_sh.at[pl.ds(s * pt + t * CH, CH)])
        pltpu.sync_copy(ibuf_v, xs2_hbm.at[pl.ds(c * np_ + s * pt + t * CH, CH)])
    plsc.subcore_barrier()

    edge_sweep(xs2_hbm)
    plsc.subcore_barrier()
    for t in range(pt // CH):
        pltpu.sync_copy(acc_sh.at[pl.ds(s * pt + t * CH, CH)], ibuf_v)
        pltpu.sync_copy(ibuf_v, out_hbm.at[c, pl.ds(s * pt + t * CH, CH)])


def _tc1_body(x_ref, w_ref, d_ref, o_ref):
    o_ref[0] = jnp.dot(x_ref[...], w_ref[...],
                       preferred_element_type=jnp.float32) * d_ref[...]


def _dis_body(np_, p_ref, o_ref):
    deg = 1.0 + jnp.sum(p_ref[...], axis=0)
    dis = lax.rsqrt(deg)
    o_ref[...] = jnp.broadcast_to(dis[:, :, None],
                                  (np_ // 128, 128, 128)).reshape(np_, 128)


def _tc3_body(inv_n, a_ref, d_ref, w_ref, b_ref, o_ref):
    i = pl.program_id(0)
    agg = jnp.concatenate([a_ref[0] * d_ref[...], a_ref[1] * d_ref[...]],
                          axis=1)
    o = jnp.dot(agg, w_ref[...], preferred_element_type=jnp.float32) + b_ref[...]
    part = jnp.sum(jnp.maximum(o, 0.0), axis=0, keepdims=True) * inv_n

    @pl.when(i == 0)
    def _():
        o_ref[...] = part

    @pl.when(i > 0)
    def _():
        o_ref[...] += part


def kernel(x, edge_index, W1, b1, W2, b2):
    n, c_in = x.shape
    e = edge_index.shape[1]
    hid = W1.shape[1]
    out_c = W2.shape[1]
    hh = hid // NC
    np_ = -(-n // (NS * 128)) * (NS * 128)
    pt = np_ // NS
    pr = pt // 128
    assert hh == 128 and e % (NW * K) == 0 and n % 8 == 0

    src = edge_index[0]
    dst = edge_index[1]
    ab = e // NS // K
    nq = -(-(e // NW) // 128)
    dst_pad = jnp.concatenate(
        [dst, jnp.full((NW * nq * 128 - e,), np_ - 1, jnp.int32)])
    dst_d = dst_pad.reshape(NW, nq, 128)
    src2 = jnp.stack([src, src + np_]).reshape(NC, NS, ab // GB, GB, K)
    dst_r = dst.reshape(NS, ab // GB, GB, K)

    mesh = plsc.VectorSubcoreMesh(core_axis_name="c", subcore_axis_name="s")

    deg_kernel = pl.kernel(
        functools.partial(_deg_body, np_, nq),
        out_type=jax.ShapeDtypeStruct((NW, 1, np_), jnp.float32),
        mesh=mesh,
        scratch_types=[
            pltpu.VMEM((nq, 128), jnp.int32),
            pltpu.VMEM((np_,), jnp.float32),
        ],
        compiler_params=pltpu.CompilerParams(needs_layout_passes=False),
    )

    gcn_kernel = pl.kernel(
        functools.partial(_gcn_body, np_, ab, hh),
        out_type=(jax.ShapeDtypeStruct((NC, np_, hh), jnp.float32),
                  jax.ShapeDtypeStruct((NC * np_, hh), jnp.float32)),
        mesh=mesh,
        scratch_types=[
            pltpu.VMEM((GB, K), jnp.int32),
            pltpu.VMEM((GB, K), jnp.int32),
            pltpu.VMEM((K, hh), jnp.float32),
            pltpu.VMEM((K, hh), jnp.float32),
            pltpu.VMEM((CH, hh), jnp.float32),
            pltpu.VMEM((CH, 128), jnp.float32),
            pltpu.VMEM((1, hh), jnp.float32),
            pltpu.SemaphoreType.DMA,
            pltpu.SemaphoreType.DMA,
            pltpu.VMEM_SHARED((np_, hh), jnp.float32),
        ],
    )

    nb = 10
    bn = n // nb

    tc1 = pl.pallas_call(
        _tc1_body,
        grid=(NC, nb),
        in_specs=[
            pl.BlockSpec((bn, c_in), lambda h, i: (i, 0)),
            pl.BlockSpec((c_in, hh), lambda h, i: (0, h)),
            pl.BlockSpec((bn, 128), lambda h, i: (i, 0)),
        ],
        out_specs=pl.BlockSpec((1, bn, hh), lambda h, i: (h, i, 0)),
        out_shape=jax.ShapeDtypeStruct((NC, np_, hh), jnp.float32),
    )

    tc_dis = pl.pallas_call(
        functools.partial(_dis_body, np_),
        grid=(1,),
        in_specs=[pl.BlockSpec((NW, np_ // 128, 128),
                               lambda i: (0, 0, 0))],
        out_specs=pl.BlockSpec((np_, 128), lambda i: (0, 0)),
        out_shape=jax.ShapeDtypeStruct((np_, 128), jnp.float32),
    )

    tc3 = pl.pallas_call(
        functools.partial(_tc3_body, 1.0 / n),
        grid=(nb,),
        in_specs=[
            pl.BlockSpec((NC, bn, hh), lambda i: (0, i, 0)),
            pl.BlockSpec((bn, 128), lambda i: (i, 0)),
            pl.BlockSpec((hid, out_c), lambda i: (0, 0)),
            pl.BlockSpec((1, out_c), lambda i: (0, 0)),
        ],
        out_specs=pl.BlockSpec((1, out_c), lambda i: (0, 0)),
        out_shape=jax.ShapeDtypeStruct((1, out_c), jnp.float32),
    )

    p = deg_kernel(dst_d).reshape(NW, np_ // 128, 128)
    dis128 = tc_dis(p)
    xs1 = tc1(x, W1, dis128)
    acc2, _ = gcn_kernel(xs1.reshape(NC * np_, hh), src2, dst_r, dis128,
                         b1.reshape(NC, 1, hh))
    return tc3(acc2, dis128, W2, b2.reshape(1, out_c))


# speedup vs baseline: 1.2773x; 1.2773x over previous
import functools

import jax
import jax.numpy as jnp
from jax import lax
from jax.experimental import pallas as pl
from jax.experimental.pallas import tpu as pltpu
from jax.experimental.pallas import tpu_sc as plsc

NC = 2
NS = 16
NW = NC * NS
K = 100
L = 16


def _deg_body(np_, nq, dst_hbm, out_hbm, idx_v, ldeg_v):
    c = lax.axis_index("c")
    s = lax.axis_index("s")
    wid = s * NC + c
    zero = jnp.zeros((L,), jnp.float32)
    ones = jnp.ones((L,), jnp.float32)

    def zr(r, _):
        ldeg_v[pl.ds(r * L, L)] = zero
        return 0

    lax.fori_loop(0, np_ // L, zr, 0)
    pltpu.sync_copy(dst_hbm.at[wid], idx_v)

    def q(i, _):
        for cb in range(128 // L):
            iv = idx_v[i, pl.ds(cb * L, L)]
            plsc.addupdate_scatter(ldeg_v, [iv], ones)
        return 0

    lax.fori_loop(0, nq, q, 0)
    pltpu.sync_copy(ldeg_v, out_hbm.at[wid, 0])


CH = 64
GB = 10


def _gcn_body(np_, ab, hh, xs1_hbm, src_hbm, dst_hbm, dis_hbm, b1_hbm,
              out_hbm, xs2_hbm,
              src_v, dst_v, rows_v, rows2_v, b1_v,
              sem, sem2, acc_sh):
    c = lax.axis_index("c")
    s = lax.axis_index("s")
    pt = np_ // NS
    ibuf_v = rows_v.at[pl.ds(0, CH)]
    dis_v = rows2_v.at[pl.ds(0, CH)]

    pltpu.sync_copy(b1_hbm.at[c], b1_v)
    for t in range(pt // CH):
        pltpu.sync_copy(xs1_hbm.at[pl.ds(c * np_ + s * pt + t * CH, CH)],
                        ibuf_v)
        pltpu.sync_copy(ibuf_v, acc_sh.at[pl.ds(s * pt + t * CH, CH)])
    plsc.subcore_barrier()

    def edge_sweep(tbl_hbm):
        bufs = (rows_v, rows2_v)
        sems = (sem, sem2)

        def grp(g, _):
            pltpu.sync_copy(src_hbm.at[c, s, g], src_v)
            pltpu.sync_copy(dst_hbm.at[s, g], dst_v)
            pend = pltpu.async_copy(tbl_hbm.at[src_v.at[0]], bufs[0], sems[0])
            for b in range(GB):
                pend.wait()
                if b + 1 < GB:
                    pend = pltpu.async_copy(tbl_hbm.at[src_v.at[b + 1]],
                                            bufs[(b + 1) % 2], sems[(b + 1) % 2])
                pltpu.sync_copy(bufs[b % 2], acc_sh.at[dst_v.at[b]], add=True)
            return 0

        lax.fori_loop(0, ab // GB, grp, 0)

    edge_sweep(xs1_hbm)
    plsc.subcore_barrier()

    for t in range(pt // CH):
        pltpu.sync_copy(acc_sh.at[pl.ds(s * pt + t * CH, CH)], ibuf_v)
        pltpu.sync_copy(dis_hbm.at[pl.ds(s * pt + t * CH, CH)], dis_v)

        def row(r, _):
            d = rows2_v[r, pl.ds(0, L)]
            for cb in range(hh // L):
                v = rows_v[r, pl.ds(cb * L, L)]
                b = b1_v[0, pl.ds(cb * L, L)]
                rows_v[r, pl.ds(cb * L, L)] = jnp.maximum(d * (d * v + b), 0.0)
            return 0

        lax.fori_loop(0, CH, row, 0)
        pltpu.sync_copy(ibuf_v, acc_sh.at[pl.ds(s * pt + t * CH, CH)])
        pltpu.sync_copy(ibuf_v, xs2_hbm.at[pl.ds(c * np_ + s * pt + t * CH, CH)])
    plsc.subcore_barrier()

    edge_sweep(xs2_hbm)
    plsc.subcore_barrier()
    for t in range(pt // CH):
        pltpu.sync_copy(acc_sh.at[pl.ds(s * pt + t * CH, CH)], ibuf_v)
        pltpu.sync_copy(ibuf_v, out_hbm.at[c, pl.ds(s * pt + t * CH, CH)])


def _tc1_body(x_ref, w_ref, d_ref, o_ref):
    o_ref[0] = jnp.dot(x_ref[...], w_ref[...],
                       preferred_element_type=jnp.float32) * d_ref[...]


def _dis_body(np_, p_ref, o_ref):
    deg = 1.0 + jnp.sum(p_ref[...], axis=0)
    dis = lax.rsqrt(deg)
    o_ref[...] = jnp.broadcast_to(dis[:, :, None],
                                  (np_ // 128, 128, 128)).reshape(np_, 128)


def _tc3_body(inv_n, a_ref, d_ref, w_ref, b_ref, o_ref):
    i = pl.program_id(0)
    agg = jnp.concatenate([a_ref[0] * d_ref[...], a_ref[1] * d_ref[...]],
                          axis=1)
    o = jnp.dot(agg, w_ref[...], preferred_element_type=jnp.float32) + b_ref[...]
    part = jnp.sum(jnp.maximum(o, 0.0), axis=0, keepdims=True) * inv_n

    @pl.when(i == 0)
    def _():
        o_ref[...] = part

    @pl.when(i > 0)
    def _():
        o_ref[...] += part


def kernel(x, edge_index, W1, b1, W2, b2):
    n, c_in = x.shape
    e = edge_index.shape[1]
    hid = W1.shape[1]
    out_c = W2.shape[1]
    hh = hid // NC
    np_ = -(-n // (NS * 128)) * (NS * 128)
    pt = np_ // NS
    pr = pt // 128
    assert hh == 128 and e % (NW * K) == 0 and n % 8 == 0

    src = edge_index[0]
    dst = edge_index[1]
    ab = e // NS // K
    nq = -(-(e // NW) // 128)
    dst_pad = jnp.concatenate(
        [dst, jnp.full((NW * nq * 128 - e,), np_ - 1, jnp.int32)])
    dst_d = dst_pad.reshape(NW, nq, 128)
    src2 = jnp.stack([src, src + np_]).reshape(NC, NS, ab // GB, GB, K)
    dst_r = dst.reshape(NS, ab // GB, GB, K)

    mesh = plsc.VectorSubcoreMesh(core_axis_name="c", subcore_axis_name="s")

    deg_kernel = pl.kernel(
        functools.partial(_deg_body, np_, nq),
        out_type=jax.ShapeDtypeStruct((NW, 1, np_), jnp.float32),
        mesh=mesh,
        scratch_types=[
            pltpu.VMEM((nq, 128), jnp.int32),
            pltpu.VMEM((np_,), jnp.float32),
        ],
        compiler_params=pltpu.CompilerParams(needs_layout_passes=False),
    )

    gcn_kernel = pl.kernel(
        functools.partial(_gcn_body, np_, ab, hh),
        out_type=(jax.ShapeDtypeStruct((NC, np_, hh), jnp.float32),
                  jax.ShapeDtypeStruct((NC * np_, hh), jnp.float32)),
        mesh=mesh,
        scratch_types=[
            pltpu.VMEM((GB, K), jnp.int32),
            pltpu.VMEM((GB, K), jnp.int32),
            pltpu.VMEM((K, hh), jnp.float32),
            pltpu.VMEM((K, hh), jnp.float32),
            pltpu.VMEM((1, hh), jnp.float32),
            pltpu.SemaphoreType.DMA,
            pltpu.SemaphoreType.DMA,
            pltpu.VMEM_SHARED((np_, hh), jnp.float32),
        ],
    )

    nb = 10
    bn = n // nb

    tc1 = pl.pallas_call(
        _tc1_body,
        grid=(NC, nb),
        in_specs=[
            pl.BlockSpec((bn, c_in), lambda h, i: (i, 0)),
            pl.BlockSpec((c_in, hh), lambda h, i: (0, h)),
            pl.BlockSpec((bn, 128), lambda h, i: (i, 0)),
        ],
        out_specs=pl.BlockSpec((1, bn, hh), lambda h, i: (h, i, 0)),
        out_shape=jax.ShapeDtypeStruct((NC, np_, hh), jnp.float32),
    )

    tc_dis = pl.pallas_call(
        functools.partial(_dis_body, np_),
        grid=(1,),
        in_specs=[pl.BlockSpec((NW, np_ // 128, 128),
                               lambda i: (0, 0, 0))],
        out_specs=pl.BlockSpec((np_, 128), lambda i: (0, 0)),
        out_shape=jax.ShapeDtypeStruct((np_, 128), jnp.float32),
    )

    tc3 = pl.pallas_call(
        functools.partial(_tc3_body, 1.0 / n),
        grid=(nb,),
        in_specs=[
            pl.BlockSpec((NC, bn, hh), lambda i: (0, i, 0)),
            pl.BlockSpec((bn, 128), lambda i: (i, 0)),
            pl.BlockSpec((hid, out_c), lambda i: (0, 0)),
            pl.BlockSpec((1, out_c), lambda i: (0, 0)),
        ],
        out_specs=pl.BlockSpec((1, out_c), lambda i: (0, 0)),
        out_shape=jax.ShapeDtypeStruct((1, out_c), jnp.float32),
    )

    p = deg_kernel(dst_d).reshape(NW, np_ // 128, 128)
    dis128 = tc_dis(p)
    xs1 = tc1(x, W1, dis128)
    acc2, _ = gcn_kernel(xs1.reshape(NC * np_, hh), src2, dst_r, dis128,
                         b1.reshape(NC, 1, hh))
    return tc3(acc2, dis128, W2, b2.reshape(1, out_c))


# speedup vs baseline: 1.3575x; 1.0627x over previous
import functools

import jax
import jax.numpy as jnp
from jax import lax
from jax.experimental import pallas as pl
from jax.experimental.pallas import tpu as pltpu
from jax.experimental.pallas import tpu_sc as plsc

NC = 2
NS = 16
NW = NC * NS
K = 100
L = 16


def _deg_body(np_, nq, dst_hbm, out_hbm, idx_v, ldeg_v):
    c = lax.axis_index("c")
    s = lax.axis_index("s")
    wid = s * NC + c
    zero = jnp.zeros((L,), jnp.float32)
    ones = jnp.ones((L,), jnp.float32)

    def zr(r, _):
        ldeg_v[pl.ds(r * L, L)] = zero
        return 0

    lax.fori_loop(0, np_ // L, zr, 0)
    pltpu.sync_copy(dst_hbm.at[wid], idx_v)

    def q(i, _):
        for cb in range(128 // L):
            iv = idx_v[i, pl.ds(cb * L, L)]
            plsc.addupdate_scatter(ldeg_v, [iv], ones)
        return 0

    lax.fori_loop(0, nq, q, 0)
    pltpu.sync_copy(ldeg_v, out_hbm.at[wid, 0])


CH = 64
GB = 10


def _gcn_body(np_, ab, hh, xs1_hbm, src_hbm, dst_hbm, dis_hbm, b1_hbm,
              out_hbm, xs2_hbm,
              src_v, dst_v, rows_v, rows2_v, b1_v,
              sem, sem2, sem3, sem4, sem5, sem6, acc_sh):
    c = lax.axis_index("c")
    s = lax.axis_index("s")
    pt = np_ // NS
    ibuf_v = rows_v.at[pl.ds(0, CH)]
    dis_v = rows2_v.at[pl.ds(0, CH)]

    pltpu.sync_copy(b1_hbm.at[c], b1_v)
    for t in range(pt // CH):
        pltpu.sync_copy(xs1_hbm.at[pl.ds(c * np_ + s * pt + t * CH, CH)],
                        ibuf_v)
        pltpu.sync_copy(ibuf_v, acc_sh.at[pl.ds(s * pt + t * CH, CH)])
    plsc.subcore_barrier()

    def edge_sweep(tbl_hbm):
        ng = ab // GB
        bufs = (rows_v, rows2_v)
        gsems = (sem, sem2)
        ssems = (sem3, sem4)

        pltpu.sync_copy(src_hbm.at[c, s, 0], src_v.at[0])
        pltpu.sync_copy(dst_hbm.at[s, 0], dst_v.at[0])
        idx_pend = None
        if ng > 1:
            idx_pend = (
                pltpu.async_copy(src_hbm.at[c, s, 1], src_v.at[1], sem5),
                pltpu.async_copy(dst_hbm.at[s, 1], dst_v.at[1], sem6),
            )
        g_pend = [None, None]
        s_pend = [None, None]
        g_pend[0] = pltpu.async_copy(tbl_hbm.at[src_v.at[0, 0]],
                                     bufs[0], gsems[0])
        for j in range(ab):
            b = j % 2
            g, bi = j // GB, j % GB
            g_pend[b].wait()
            if j + 1 < ab:
                b2 = (j + 1) % 2
                g2, bi2 = (j + 1) // GB, (j + 1) % GB
                if bi2 == 0:
                    idx_pend[0].wait()
                    idx_pend[1].wait()
                    if g2 + 1 < ng:
                        sl = (g2 + 1) % 3
                        idx_pend = (
                            pltpu.async_copy(src_hbm.at[c, s, g2 + 1],
                                             src_v.at[sl], sem5),
                            pltpu.async_copy(dst_hbm.at[s, g2 + 1],
                                             dst_v.at[sl], sem6),
                        )
                if s_pend[b2] is not None:
                    s_pend[b2].wait()
                g_pend[b2] = pltpu.async_copy(
                    tbl_hbm.at[src_v.at[g2 % 3, bi2]], bufs[b2], gsems[b2])
            s_pend[b] = pltpu.async_copy(
                bufs[b], acc_sh.at[dst_v.at[g % 3, bi]], ssems[b], add=True)
        s_pend[(ab - 2) % 2].wait()
        s_pend[(ab - 1) % 2].wait()

    edge_sweep(xs1_hbm)
    plsc.subcore_barrier()

    for t in range(pt // CH):
        pltpu.sync_copy(acc_sh.at[pl.ds(s * pt + t * CH, CH)], ibuf_v)
        pltpu.sync_copy(dis_hbm.at[pl.ds(s * pt + t * CH, CH)], dis_v)

        def row(r, _):
            d = rows2_v[r, pl.ds(0, L)]
            for cb in range(hh // L):
                v = rows_v[r, pl.ds(cb * L, L)]
                b = b1_v[0, pl.ds(cb * L, L)]
                rows_v[r, pl.ds(cb * L, L)] = jnp.maximum(d * (d * v + b), 0.0)
            return 0

        lax.fori_loop(0, CH, row, 0)
        pltpu.sync_copy(ibuf_v, acc_sh.at[pl.ds(s * pt + t * CH, CH)])
        pltpu.sync_copy(ibuf_v, xs2_hbm.at[pl.ds(c * np_ + s * pt + t * CH, CH)])
    plsc.subcore_barrier()

    edge_sweep(xs2_hbm)
    plsc.subcore_barrier()
    for t in range(pt // CH):
        pltpu.sync_copy(acc_sh.at[pl.ds(s * pt + t * CH, CH)], ibuf_v)
        pltpu.sync_copy(ibuf_v, out_hbm.at[c, pl.ds(s * pt + t * CH, CH)])


def _tc1_body(x_ref, w_ref, d_ref, o_ref):
    o_ref[0] = jnp.dot(x_ref[...], w_ref[...],
                       preferred_element_type=jnp.float32) * d_ref[...]


def _dis_body(np_, p_ref, o_ref):
    deg = 1.0 + jnp.sum(p_ref[...], axis=0)
    dis = lax.rsqrt(deg)
    o_ref[...] = jnp.broadcast_to(dis[:, :, None],
                                  (np_ // 128, 128, 128)).reshape(np_, 128)


def _tc3_body(inv_n, a_ref, d_ref, w_ref, b_ref, o_ref):
    i = pl.program_id(0)
    agg = jnp.concatenate([a_ref[0] * d_ref[...], a_ref[1] * d_ref[...]],
                          axis=1)
    o = jnp.dot(agg, w_ref[...], preferred_element_type=jnp.float32) + b_ref[...]
    part = jnp.sum(jnp.maximum(o, 0.0), axis=0, keepdims=True) * inv_n

    @pl.when(i == 0)
    def _():
        o_ref[...] = part

    @pl.when(i > 0)
    def _():
        o_ref[...] += part


def kernel(x, edge_index, W1, b1, W2, b2):
    n, c_in = x.shape
    e = edge_index.shape[1]
    hid = W1.shape[1]
    out_c = W2.shape[1]
    hh = hid // NC
    np_ = -(-n // (NS * 128)) * (NS * 128)
    pt = np_ // NS
    pr = pt // 128
    assert hh == 128 and e % (NW * K) == 0 and n % 8 == 0

    src = edge_index[0]
    dst = edge_index[1]
    ab = e // NS // K
    nq = -(-(e // NW) // 128)
    dst_pad = jnp.concatenate(
        [dst, jnp.full((NW * nq * 128 - e,), np_ - 1, jnp.int32)])
    dst_d = dst_pad.reshape(NW, nq, 128)
    src2 = jnp.stack([src, src + np_]).reshape(NC, NS, ab // GB, GB, K)
    dst_r = dst.reshape(NS, ab // GB, GB, K)

    mesh = plsc.VectorSubcoreMesh(core_axis_name="c", subcore_axis_name="s")

    deg_kernel = pl.kernel(
        functools.partial(_deg_body, np_, nq),
        out_type=jax.ShapeDtypeStruct((NW, 1, np_), jnp.float32),
        mesh=mesh,
        scratch_types=[
            pltpu.VMEM((nq, 128), jnp.int32),
            pltpu.VMEM((np_,), jnp.float32),
        ],
        compiler_params=pltpu.CompilerParams(needs_layout_passes=False),
    )

    gcn_kernel = pl.kernel(
        functools.partial(_gcn_body, np_, ab, hh),
        out_type=(jax.ShapeDtypeStruct((NC, np_, hh), jnp.float32),
                  jax.ShapeDtypeStruct((NC * np_, hh), jnp.float32)),
        mesh=mesh,
        scratch_types=[
            pltpu.VMEM((3, GB, K), jnp.int32),
            pltpu.VMEM((3, GB, K), jnp.int32),
            pltpu.VMEM((K, hh), jnp.float32),
            pltpu.VMEM((K, hh), jnp.float32),
            pltpu.VMEM((1, hh), jnp.float32),
            pltpu.SemaphoreType.DMA,
            pltpu.SemaphoreType.DMA,
            pltpu.SemaphoreType.DMA,
            pltpu.SemaphoreType.DMA,
            pltpu.SemaphoreType.DMA,
            pltpu.SemaphoreType.DMA,
            pltpu.VMEM_SHARED((np_, hh), jnp.float32),
        ],
    )

    nb = 10
    bn = n // nb

    tc1 = pl.pallas_call(
        _tc1_body,
        grid=(NC, nb),
        in_specs=[
            pl.BlockSpec((bn, c_in), lambda h, i: (i, 0)),
            pl.BlockSpec((c_in, hh), lambda h, i: (0, h)),
            pl.BlockSpec((bn, 128), lambda h, i: (i, 0)),
        ],
        out_specs=pl.BlockSpec((1, bn, hh), lambda h, i: (h, i, 0)),
        out_shape=jax.ShapeDtypeStruct((NC, np_, hh), jnp.float32),
    )

    tc_dis = pl.pallas_call(
        functools.partial(_dis_body, np_),
        grid=(1,),
        in_specs=[pl.BlockSpec((NW, np_ // 128, 128),
                               lambda i: (0, 0, 0))],
        out_specs=pl.BlockSpec((np_, 128), lambda i: (0, 0)),
        out_shape=jax.ShapeDtypeStruct((np_, 128), jnp.float32),
    )

    tc3 = pl.pallas_call(
        functools.partial(_tc3_body, 1.0 / n),
        grid=(nb,),
        in_specs=[
            pl.BlockSpec((NC, bn, hh), lambda i: (0, i, 0)),
            pl.BlockSpec((bn, 128), lambda i: (i, 0)),
            pl.BlockSpec((hid, out_c), lambda i: (0, 0)),
            pl.BlockSpec((1, out_c), lambda i: (0, 0)),
        ],
        out_specs=pl.BlockSpec((1, out_c), lambda i: (0, 0)),
        out_shape=jax.ShapeDtypeStruct((1, out_c), jnp.float32),
    )

    p = deg_kernel(dst_d).reshape(NW, np_ // 128, 128)
    dis128 = tc_dis(p)
    xs1 = tc1(x, W1, dis128)
    acc2, _ = gcn_kernel(xs1.reshape(NC * np_, hh), src2, dst_r, dis128,
                         b1.reshape(NC, 1, hh))
    return tc3(acc2, dis128, W2, b2.reshape(1, out_c))


# speedup vs baseline: 1.3658x; 1.0062x over previous
import functools

import jax
import jax.numpy as jnp
from jax import lax
from jax.experimental import pallas as pl
from jax.experimental.pallas import tpu as pltpu
from jax.experimental.pallas import tpu_sc as plsc

NC = 2
NS = 16
NW = NC * NS
K = 100
L = 16


def _deg_body(np_, nq, dst_hbm, out_hbm, idx_v, ldeg_v):
    c = lax.axis_index("c")
    s = lax.axis_index("s")
    wid = s * NC + c
    zero = jnp.zeros((L,), jnp.float32)
    ones = jnp.ones((L,), jnp.float32)

    def zr(r, _):
        ldeg_v[pl.ds(r * L, L)] = zero
        return 0

    lax.fori_loop(0, np_ // L, zr, 0)
    pltpu.sync_copy(dst_hbm.at[wid], idx_v)

    def q(i, _):
        for cb in range(128 // L):
            iv = idx_v[i, pl.ds(cb * L, L)]
            plsc.addupdate_scatter(ldeg_v, [iv], ones)
        return 0

    lax.fori_loop(0, nq, q, 0)
    pltpu.sync_copy(ldeg_v, out_hbm.at[wid, 0])


CH = 64
GB = 10


def _gcn_body(np_, ab, hh, xs1_hbm, src_hbm, dst_hbm, dis_hbm, b1_hbm,
              out_hbm, xs2_hbm,
              src_v, dst_v, rows_v, rows2_v, b1_v,
              sem, sem2, sem3, sem4, sem5, sem6, acc_sh):
    c = lax.axis_index("c")
    s = lax.axis_index("s")
    pt = np_ // NS
    ibuf_v = rows_v.at[pl.ds(0, CH)]
    dis_v = rows2_v.at[pl.ds(0, CH)]

    pltpu.sync_copy(b1_hbm.at[c], b1_v)
    for t in range(pt // CH):
        pltpu.sync_copy(xs1_hbm.at[pl.ds(c * np_ + s * pt + t * CH, CH)],
                        ibuf_v)
        pltpu.sync_copy(ibuf_v, acc_sh.at[pl.ds(s * pt + t * CH, CH)])
    plsc.subcore_barrier()

    def edge_sweep(tbl_hbm):
        ng = ab // GB
        bufs = (rows_v, rows2_v)
        gsems = (sem, sem2)
        ssems = (sem3, sem4)

        pltpu.sync_copy(src_hbm.at[c, s, 0], src_v.at[0])
        pltpu.sync_copy(dst_hbm.at[s, 0], dst_v.at[0])
        idx_pend = None
        if ng > 1:
            idx_pend = (
                pltpu.async_copy(src_hbm.at[c, s, 1], src_v.at[1], sem5),
                pltpu.async_copy(dst_hbm.at[s, 1], dst_v.at[1], sem6),
            )
        g_pend = [None, None]
        s_pend = [None, None]
        g_pend[0] = pltpu.async_copy(tbl_hbm.at[src_v.at[0, 0]],
                                     bufs[0], gsems[0])
        for j in range(ab):
            b = j % 2
            g, bi = j // GB, j % GB
            g_pend[b].wait()
            if j + 1 < ab:
                b2 = (j + 1) % 2
                g2, bi2 = (j + 1) // GB, (j + 1) % GB
                if bi2 == 0:
                    idx_pend[0].wait()
                    idx_pend[1].wait()
                    if g2 + 1 < ng:
                        sl = (g2 + 1) % 3
                        idx_pend = (
                            pltpu.async_copy(src_hbm.at[c, s, g2 + 1],
                                             src_v.at[sl], sem5),
                            pltpu.async_copy(dst_hbm.at[s, g2 + 1],
                                             dst_v.at[sl], sem6),
                        )
                if s_pend[b2] is not None:
                    s_pend[b2].wait()
                g_pend[b2] = pltpu.async_copy(
                    tbl_hbm.at[src_v.at[g2 % 3, bi2]], bufs[b2], gsems[b2])
            s_pend[b] = pltpu.async_copy(
                bufs[b], acc_sh.at[dst_v.at[g % 3, bi]], ssems[b], add=True)
        s_pend[(ab - 2) % 2].wait()
        s_pend[(ab - 1) % 2].wait()

    edge_sweep(xs1_hbm)
    plsc.subcore_barrier()

    for t in range(pt // CH):
        pltpu.sync_copy(acc_sh.at[pl.ds(s * pt + t * CH, CH)], ibuf_v)
        pltpu.sync_copy(dis_hbm.at[pl.ds(s * pt + t * CH, CH)], dis_v)

        def row(r, _):
            d = rows2_v[r, pl.ds(0, L)]
            for cb in range(hh // L):
                v = rows_v[r, pl.ds(cb * L, L)]
                b = b1_v[0, pl.ds(cb * L, L)]
                rows_v[r, pl.ds(cb * L, L)] = jnp.maximum(d * (d * v + b), 0.0)
            return 0

        lax.fori_loop(0, CH, row, 0)
        pltpu.sync_copy(ibuf_v, acc_sh.at[pl.ds(s * pt + t * CH, CH)])
        pltpu.sync_copy(ibuf_v, xs2_hbm.at[pl.ds(c * np_ + s * pt + t * CH, CH)])
    plsc.subcore_barrier()

    edge_sweep(xs2_hbm)
    plsc.subcore_barrier()
    for t in range(pt // CH):
        pltpu.sync_copy(acc_sh.at[pl.ds(s * pt + t * CH, CH)], ibuf_v)
        pltpu.sync_copy(ibuf_v, out_hbm.at[c, pl.ds(s * pt + t * CH, CH)])


def _tc1_body(bn, x_ref, w_ref, p_ref, o_ref, d_ref):
    deg = 1.0 + jnp.sum(p_ref[...], axis=0)
    dis = lax.rsqrt(deg)
    disb = jnp.broadcast_to(dis[:, :, None],
                            (bn // 128, 128, 128)).reshape(bn, 128)
    d_ref[...] = disb
    o_ref[0] = jnp.dot(x_ref[...], w_ref[...],
                       preferred_element_type=jnp.float32) * disb


def _tc3_body(inv_n, a_ref, d_ref, w_ref, b_ref, o_ref):
    i = pl.program_id(0)
    agg = jnp.concatenate([a_ref[0] * d_ref[...], a_ref[1] * d_ref[...]],
                          axis=1)
    o = jnp.dot(agg, w_ref[...], preferred_element_type=jnp.float32) + b_ref[...]
    part = jnp.sum(jnp.maximum(o, 0.0), axis=0, keepdims=True) * inv_n

    @pl.when(i == 0)
    def _():
        o_ref[...] = part

    @pl.when(i > 0)
    def _():
        o_ref[...] += part


def kernel(x, edge_index, W1, b1, W2, b2):
    n, c_in = x.shape
    e = edge_index.shape[1]
    hid = W1.shape[1]
    out_c = W2.shape[1]
    hh = hid // NC
    np_ = -(-n // (NS * 128)) * (NS * 128)
    pt = np_ // NS
    pr = pt // 128
    assert hh == 128 and e % (NW * K) == 0 and n % 8 == 0

    src = edge_index[0]
    dst = edge_index[1]
    ab = e // NS // K
    nq = -(-(e // NW) // 128)
    dst_pad = jnp.concatenate(
        [dst, jnp.full((NW * nq * 128 - e,), np_ - 1, jnp.int32)])
    dst_d = dst_pad.reshape(NW, nq, 128)
    src2 = jnp.stack([src, src + np_]).reshape(NC, NS, ab // GB, GB, K)
    dst_r = dst.reshape(NS, ab // GB, GB, K)

    mesh = plsc.VectorSubcoreMesh(core_axis_name="c", subcore_axis_name="s")

    deg_kernel = pl.kernel(
        functools.partial(_deg_body, np_, nq),
        out_type=jax.ShapeDtypeStruct((NW, 1, np_), jnp.float32),
        mesh=mesh,
        scratch_types=[
            pltpu.VMEM((nq, 128), jnp.int32),
            pltpu.VMEM((np_,), jnp.float32),
        ],
        compiler_params=pltpu.CompilerParams(needs_layout_passes=False),
    )

    gcn_kernel = pl.kernel(
        functools.partial(_gcn_body, np_, ab, hh),
        out_type=(jax.ShapeDtypeStruct((NC, np_, hh), jnp.float32),
                  jax.ShapeDtypeStruct((NC * np_, hh), jnp.float32)),
        mesh=mesh,
        scratch_types=[
            pltpu.VMEM((3, GB, K), jnp.int32),
            pltpu.VMEM((3, GB, K), jnp.int32),
            pltpu.VMEM((K, hh), jnp.float32),
            pltpu.VMEM((K, hh), jnp.float32),
            pltpu.VMEM((1, hh), jnp.float32),
            pltpu.SemaphoreType.DMA,
            pltpu.SemaphoreType.DMA,
            pltpu.SemaphoreType.DMA,
            pltpu.SemaphoreType.DMA,
            pltpu.SemaphoreType.DMA,
            pltpu.SemaphoreType.DMA,
            pltpu.VMEM_SHARED((np_, hh), jnp.float32),
        ],
    )

    nb = 10
    bn = n // nb
    bn1 = 1024
    nb1 = np_ // bn1

    tc1 = pl.pallas_call(
        functools.partial(_tc1_body, bn1),
        grid=(NC, nb1),
        in_specs=[
            pl.BlockSpec((bn1, c_in), lambda h, i: (i, 0)),
            pl.BlockSpec((c_in, hh), lambda h, i: (0, h)),
            pl.BlockSpec((NW, bn1 // 128, 128), lambda h, i: (0, i, 0)),
        ],
        out_specs=(pl.BlockSpec((1, bn1, hh), lambda h, i: (h, i, 0)),
                   pl.BlockSpec((bn1, 128), lambda h, i: (i, 0))),
        out_shape=(jax.ShapeDtypeStruct((NC, np_, hh), jnp.float32),
                   jax.ShapeDtypeStruct((np_, 128), jnp.float32)),
    )

    tc3 = pl.pallas_call(
        functools.partial(_tc3_body, 1.0 / n),
        grid=(nb,),
        in_specs=[
            pl.BlockSpec((NC, bn, hh), lambda i: (0, i, 0)),
            pl.BlockSpec((bn, 128), lambda i: (i, 0)),
            pl.BlockSpec((hid, out_c), lambda i: (0, 0)),
            pl.BlockSpec((1, out_c), lambda i: (0, 0)),
        ],
        out_specs=pl.BlockSpec((1, out_c), lambda i: (0, 0)),
        out_shape=jax.ShapeDtypeStruct((1, out_c), jnp.float32),
    )

    x_pad = jnp.pad(x, ((0, np_ - n), (0, 0)))
    p = deg_kernel(dst_d).reshape(NW, np_ // 128, 128)
    xs1, dis128 = tc1(x_pad, W1, p)
    acc2, _ = gcn_kernel(xs1.reshape(NC * np_, hh), src2, dst_r, dis128,
                         b1.reshape(NC, 1, hh))
    return tc3(acc2, dis128, W2, b2.reshape(1, out_c))


# speedup vs baseline: 1.3665x; 1.0005x over previous
import functools

import jax
import jax.numpy as jnp
from jax import lax
from jax.experimental import pallas as pl
from jax.experimental.pallas import tpu as pltpu
from jax.experimental.pallas import tpu_sc as plsc

NC = 2
NS = 16
NW = NC * NS
K = 100
L = 16


def _deg_body(np_, nq, dst_hbm, out_hbm, idx_v, ldeg_v):
    c = lax.axis_index("c")
    s = lax.axis_index("s")
    wid = s * NC + c
    zero = jnp.zeros((L,), jnp.float32)
    ones = jnp.ones((L,), jnp.float32)

    def zr(r, _):
        ldeg_v[pl.ds(r * L, L)] = zero
        return 0

    lax.fori_loop(0, np_ // L, zr, 0)
    pltpu.sync_copy(dst_hbm.at[wid], idx_v)

    def q(i, _):
        for cb in range(128 // L):
            iv = idx_v[i, pl.ds(cb * L, L)]
            plsc.addupdate_scatter(ldeg_v, [iv], ones)
        return 0

    lax.fori_loop(0, nq, q, 0)
    pltpu.sync_copy(ldeg_v, out_hbm.at[wid, 0])


CH = 64
GB = 10


def _gcn_body(np_, ab, hh, xs1_hbm, src_hbm, dst_hbm, dis_hbm, b1_hbm,
              out_hbm, xs2_hbm,
              src_v, dst_v, rows_v, rows2_v, b1_v,
              sem, sem2, sem3, sem4, sem5, sem6, acc_sh):
    c = lax.axis_index("c")
    s = lax.axis_index("s")
    pt = np_ // NS
    ibuf_v = rows_v.at[pl.ds(0, CH)]
    dis_v = rows2_v.at[pl.ds(0, CH)]

    pltpu.sync_copy(b1_hbm.at[c], b1_v)
    for t in range(pt // CH):
        pltpu.sync_copy(xs1_hbm.at[pl.ds(c * np_ + s * pt + t * CH, CH)],
                        ibuf_v)
        pltpu.sync_copy(ibuf_v, acc_sh.at[pl.ds(s * pt + t * CH, CH)])
    plsc.subcore_barrier()

    def edge_sweep(tbl_hbm):
        ng = ab // GB
        bufs = (rows_v, rows2_v)
        gsems = (sem, sem2)
        ssems = (sem3, sem4)

        pltpu.sync_copy(src_hbm.at[c, s, 0], src_v.at[0])
        pltpu.sync_copy(dst_hbm.at[s, 0], dst_v.at[0])
        idx_pend = None
        if ng > 1:
            idx_pend = (
                pltpu.async_copy(src_hbm.at[c, s, 1], src_v.at[1], sem5),
                pltpu.async_copy(dst_hbm.at[s, 1], dst_v.at[1], sem6),
            )
        g_pend = [None, None]
        s_pend = [None, None]
        g_pend[0] = pltpu.async_copy(tbl_hbm.at[src_v.at[0, 0]],
                                     bufs[0], gsems[0])
        for j in range(ab):
            b = j % 2
            g, bi = j // GB, j % GB
            g_pend[b].wait()
            if j + 1 < ab:
                b2 = (j + 1) % 2
                g2, bi2 = (j + 1) // GB, (j + 1) % GB
                if bi2 == 0:
                    idx_pend[0].wait()
                    idx_pend[1].wait()
                    if g2 + 1 < ng:
                        sl = (g2 + 1) % 3
                        idx_pend = (
                            pltpu.async_copy(src_hbm.at[c, s, g2 + 1],
                                             src_v.at[sl], sem5),
                            pltpu.async_copy(dst_hbm.at[s, g2 + 1],
                                             dst_v.at[sl], sem6),
                        )
                if s_pend[b2] is not None:
                    s_pend[b2].wait()
                g_pend[b2] = pltpu.async_copy(
                    tbl_hbm.at[src_v.at[g2 % 3, bi2]], bufs[b2], gsems[b2])
            s_pend[b] = pltpu.async_copy(
                bufs[b], acc_sh.at[dst_v.at[g % 3, bi]], ssems[b], add=True)
        s_pend[(ab - 2) % 2].wait()
        s_pend[(ab - 1) % 2].wait()

    edge_sweep(xs1_hbm)
    plsc.subcore_barrier()

    for t in range(pt // CH):
        pltpu.sync_copy(acc_sh.at[pl.ds(s * pt + t * CH, CH)], ibuf_v)
        pltpu.sync_copy(dis_hbm.at[pl.ds(s * pt + t * CH, CH)], dis_v)

        def row(r, _):
            d = rows2_v[r, pl.ds(0, L)]
            for cb in range(hh // L):
                v = rows_v[r, pl.ds(cb * L, L)]
                b = b1_v[0, pl.ds(cb * L, L)]
                rows_v[r, pl.ds(cb * L, L)] = jnp.maximum(d * (d * v + b), 0.0)
            return 0

        lax.fori_loop(0, CH, row, 0)
        pltpu.sync_copy(ibuf_v, acc_sh.at[pl.ds(s * pt + t * CH, CH)])
        pltpu.sync_copy(ibuf_v, xs2_hbm.at[pl.ds(c * np_ + s * pt + t * CH, CH)])
    plsc.subcore_barrier()

    edge_sweep(xs2_hbm)
    plsc.subcore_barrier()
    for t in range(pt // CH):
        pltpu.sync_copy(acc_sh.at[pl.ds(s * pt + t * CH, CH)], ibuf_v)
        pltpu.sync_copy(ibuf_v, out_hbm.at[c, pl.ds(s * pt + t * CH, CH)])


def _tc1_body(bn, x_ref, w_ref, p_ref, o_ref, d_ref):
    deg = 1.0 + jnp.sum(p_ref[...], axis=0)
    dis = lax.rsqrt(deg)
    disb = jnp.broadcast_to(dis[:, :, None],
                            (bn // 128, 128, 128)).reshape(bn, 128)
    d_ref[...] = disb
    o_ref[0] = jnp.dot(x_ref[...], w_ref[...],
                       preferred_element_type=jnp.float32) * disb


def _tc3_body(inv_n, a_ref, d_ref, w_ref, b_ref, o_ref):
    i = pl.program_id(0)
    agg = jnp.concatenate([a_ref[0] * d_ref[...], a_ref[1] * d_ref[...]],
                          axis=1)
    o = jnp.dot(agg, w_ref[...], preferred_element_type=jnp.float32) + b_ref[...]
    part = jnp.sum(jnp.maximum(o, 0.0), axis=0, keepdims=True) * inv_n

    @pl.when(i == 0)
    def _():
        o_ref[...] = part

    @pl.when(i > 0)
    def _():
        o_ref[...] += part


def kernel(x, edge_index, W1, b1, W2, b2):
    n, c_in = x.shape
    e = edge_index.shape[1]
    hid = W1.shape[1]
    out_c = W2.shape[1]
    hh = hid // NC
    np_ = -(-n // (NS * 128)) * (NS * 128)
    pt = np_ // NS
    assert hh == 128 and e % (NW * K) == 0 and n % 8 == 0

    src = edge_index[0]
    dst = edge_index[1]
    ab = e // NS // K
    nq = -(-(e // NW) // 128)
    dst_pad = jnp.concatenate(
        [dst, jnp.full((NW * nq * 128 - e,), np_ - 1, jnp.int32)])
    dst_d = dst_pad.reshape(NW, nq, 128)
    src2 = jnp.stack([src, src + np_]).reshape(NC, NS, ab // GB, GB, K)
    dst_r = dst.reshape(NS, ab // GB, GB, K)

    mesh = plsc.VectorSubcoreMesh(core_axis_name="c", subcore_axis_name="s")

    deg_kernel = pl.kernel(
        functools.partial(_deg_body, np_, nq),
        out_type=jax.ShapeDtypeStruct((NW, 1, np_), jnp.float32),
        mesh=mesh,
        scratch_types=[
            pltpu.VMEM((nq, 128), jnp.int32),
            pltpu.VMEM((np_,), jnp.float32),
        ],
        compiler_params=pltpu.CompilerParams(needs_layout_passes=False),
    )

    gcn_kernel = pl.kernel(
        functools.partial(_gcn_body, np_, ab, hh),
        out_type=(jax.ShapeDtypeStruct((NC, np_, hh), jnp.float32),
                  jax.ShapeDtypeStruct((NC * np_, hh), jnp.float32)),
        mesh=mesh,
        scratch_types=[
            pltpu.VMEM((3, GB, K), jnp.int32),
            pltpu.VMEM((3, GB, K), jnp.int32),
            pltpu.VMEM((K, hh), jnp.float32),
            pltpu.VMEM((K, hh), jnp.float32),
            pltpu.VMEM((1, hh), jnp.float32),
            pltpu.SemaphoreType.DMA,
            pltpu.SemaphoreType.DMA,
            pltpu.SemaphoreType.DMA,
            pltpu.SemaphoreType.DMA,
            pltpu.SemaphoreType.DMA,
            pltpu.SemaphoreType.DMA,
            pltpu.VMEM_SHARED((np_, hh), jnp.float32),
        ],
    )

    nb = 10
    bn = n // nb
    bn1 = 1024
    nb1 = np_ // bn1

    tc1 = pl.pallas_call(
        functools.partial(_tc1_body, bn1),
        grid=(NC, nb1),
        in_specs=[
            pl.BlockSpec((bn1, c_in), lambda h, i: (i, 0)),
            pl.BlockSpec((c_in, hh), lambda h, i: (0, h)),
            pl.BlockSpec((NW, bn1 // 128, 128), lambda h, i: (0, i, 0)),
        ],
        out_specs=(pl.BlockSpec((1, bn1, hh), lambda h, i: (h, i, 0)),
                   pl.BlockSpec((bn1, 128), lambda h, i: (i, 0))),
        out_shape=(jax.ShapeDtypeStruct((NC, np_, hh), jnp.float32),
                   jax.ShapeDtypeStruct((np_, 128), jnp.float32)),
    )

    tc3 = pl.pallas_call(
        functools.partial(_tc3_body, 1.0 / n),
        grid=(nb,),
        in_specs=[
            pl.BlockSpec((NC, bn, hh), lambda i: (0, i, 0)),
            pl.BlockSpec((bn, 128), lambda i: (i, 0)),
            pl.BlockSpec((hid, out_c), lambda i: (0, 0)),
            pl.BlockSpec((1, out_c), lambda i: (0, 0)),
        ],
        out_specs=pl.BlockSpec((1, out_c), lambda i: (0, 0)),
        out_shape=jax.ShapeDtypeStruct((1, out_c), jnp.float32),
    )

    x_pad = jnp.pad(x, ((0, np_ - n), (0, 0)))
    p = deg_kernel(dst_d).reshape(NW, np_ // 128, 128)
    xs1, dis128 = tc1(x_pad, W1, p)
    acc2, _ = gcn_kernel(xs1.reshape(NC * np_, hh), src2, dst_r, dis128,
                         b1.reshape(NC, 1, hh))
    return tc3(acc2, dis128, W2, b2.reshape(1, out_c))
